# Initial kernel scaffold; baseline (speedup 1.0000x reference)
import jax
import jax.numpy as jnp
from jax.experimental import pallas as pl


def kernel(indices, table, W, b):
    raise NotImplementedError("write your pallas kernel here")


# speedup vs baseline: 3.3431x; 3.3431x over previous
import functools

import jax
import jax.numpy as jnp
from jax import lax
from jax.experimental import pallas as pl
from jax.experimental.pallas import tpu as pltpu
from jax.experimental.pallas import tpu_sc as plsc

D_FEAT = 64
D_MODEL = 128
CHUNK = 128
NC, NS = 2, 16
NW = NC * NS
VBLK = 1024


def _tc_project_vocab(table, W, b2d):
    v = table.shape[0]
    grid = (v + VBLK - 1) // VBLK
    mb = VBLK // 128
    rs_rows = grid * mb

    def body(t_ref, w_ref, b_ref, p_ref, s_ref):
        t = t_ref[...]
        y = jnp.dot(t, w_ref[...], preferred_element_type=jnp.float32) + b_ref[...]
        p_ref[...] = jax.nn.gelu(y)
        s_ref[...] = jnp.sum(t.reshape(mb, 128, D_FEAT), axis=-1)

    return pl.pallas_call(
        body,
        grid=(grid,),
        in_specs=[
            pl.BlockSpec((VBLK, D_FEAT), lambda i: (i, 0)),
            pl.BlockSpec((D_FEAT, D_MODEL), lambda i: (0, 0)),
            pl.BlockSpec((1, D_MODEL), lambda i: (0, 0)),
        ],
        out_specs=[
            pl.BlockSpec((VBLK, D_MODEL), lambda i: (i, 0)),
            pl.BlockSpec((mb, 128), lambda i: (i, 0)),
        ],
        out_shape=[
            jax.ShapeDtypeStruct((v, D_MODEL), jnp.float32),
            jax.ShapeDtypeStruct((rs_rows, 128), jnp.float32),
        ],
    )(table, W, b2d)


def _sc_gather(idx3d, proj):
    per_w = idx3d.shape[1]
    n = NW * per_w * CHUNK
    mesh = plsc.VectorSubcoreMesh(core_axis_name="c", subcore_axis_name="s")

    @functools.partial(
        pl.kernel,
        out_type=jax.ShapeDtypeStruct((n, D_MODEL), jnp.float32),
        mesh=mesh,
        scratch_types=[
            pltpu.VMEM((per_w, CHUNK), jnp.int32),
            pltpu.VMEM((CHUNK, D_MODEL), jnp.float32),
            pltpu.SemaphoreType.DMA,
        ],
    )
    def k(idx_hbm, proj_hbm, out_hbm, idx_v, rows_v, gsem):
        wid = lax.axis_index("s") * NC + lax.axis_index("c")
        base = wid * per_w
        pltpu.sync_copy(idx_hbm.at[wid], idx_v)

        def body(j, carry):
            pltpu.async_copy(proj_hbm.at[idx_v.at[j]], rows_v, gsem).wait()
            pltpu.sync_copy(rows_v, out_hbm.at[pl.ds((base + j) * CHUNK, CHUNK)])
            return carry

        lax.fori_loop(0, per_w, body, 0)

    return k(idx3d, proj)


def _sc_mask(idx_flat, rs_flat):
    n = idx_flat.shape[0]
    per_w = n // NW
    groups = per_w // 16
    mesh = plsc.VectorSubcoreMesh(core_axis_name="c", subcore_axis_name="s")

    @functools.partial(
        pl.kernel,
        out_type=jax.ShapeDtypeStruct((n,), jnp.float32),
        mesh=mesh,
        scratch_types=[
            pltpu.VMEM(rs_flat.shape, jnp.float32),
            pltpu.VMEM((per_w,), jnp.int32),
            pltpu.VMEM((per_w,), jnp.float32),
        ],
        compiler_params=pltpu.CompilerParams(needs_layout_passes=False),
    )
    def k(idx_hbm, rs_hbm, out_hbm, rs_v, idx_v, m_v):
        wid = lax.axis_index("s") * NC + lax.axis_index("c")
        pltpu.sync_copy(rs_hbm, rs_v)
        pltpu.sync_copy(idx_hbm.at[pl.ds(wid * per_w, per_w)], idx_v)

        def body(j, carry):
            vidx = idx_v[pl.ds(j * 16, 16)]
            vals = plsc.load_gather(rs_v, [vidx])
            m_v[pl.ds(j * 16, 16)] = jnp.where(vals == 0.0, 1.0, 0.0).astype(jnp.float32)
            return carry

        lax.fori_loop(0, groups, body, 0)
        pltpu.sync_copy(m_v, out_hbm.at[pl.ds(wid * per_w, per_w)])

    return k(idx_flat, rs_flat)


def kernel(indices, table, W, b):
    bsz, seq = indices.shape
    n = bsz * seq
    idx3d = indices.reshape(NW, n // (NW * CHUNK), CHUNK).astype(jnp.int32)
    proj, rowsum = _tc_project_vocab(table, W, b.reshape(1, D_MODEL))
    out_flat = _sc_gather(idx3d, proj)
    mask_flat = _sc_mask(idx3d.reshape(n), rowsum.reshape(-1))
    out = out_flat.reshape(bsz, seq, D_MODEL)
    mask = mask_flat.reshape(bsz, seq).astype(bool)[:, None, None, :]
    return out, mask
